# Initial kernel scaffold; baseline (speedup 1.0000x reference)
import jax
import jax.numpy as jnp
from jax.experimental import pallas as pl


def kernel(queries, drug_h, keys, w_rel, Wg_w, Wg_b):
    raise NotImplementedError("write your pallas kernel here")


# speedup vs baseline: 2.4671x; 2.4671x over previous
import functools

import jax
import jax.numpy as jnp
from jax.experimental import pallas as pl
from jax.experimental.pallas import tpu as pltpu

Q_ = 1024
K_ = 100000
D_ = 128
TK_ = 8
CH_ = 128
KB_ = 2048
NB_ = 49
KPAD_ = NB_ * KB_
NCH_ = KPAD_ // CH_
CPB_ = KB_ // CH_
NEG_ = -1e30


def _sim_body(q_ref, k_ref, sim_ref, m_ref, qn_ref):
    kb = pl.program_id(0)

    @pl.when(kb == 0)
    def _():
        q = q_ref[...]
        qn = jnp.sqrt(jnp.sum(q * q, axis=1, keepdims=True))
        qn_ref[...] = q / jnp.maximum(qn, 1e-8)

    kblk = k_ref[...]
    kn = jnp.sqrt(jnp.sum(kblk * kblk, axis=1, keepdims=True))
    knorm = kblk / jnp.maximum(kn, 1e-8)
    sim = jax.lax.dot_general(qn_ref[...], knorm,
                              (((1,), (1,)), ((), ())),
                              preferred_element_type=jnp.float32)
    col = kb * KB_ + jax.lax.broadcasted_iota(jnp.int32, (Q_, KB_), 1)
    sim = jnp.where(col < K_, sim, NEG_)
    sim_ref[...] = sim
    maxes = [jnp.max(sim[:, c * CH_:(c + 1) * CH_], axis=1, keepdims=True)
             for c in range(CPB_)]
    m_ref[0] = jnp.concatenate(maxes, axis=1)


@functools.partial(jax.jit, static_argnames=())
def _sim_stage(queries, keys_pad):
    return pl.pallas_call(
        _sim_body,
        grid=(NB_,),
        in_specs=[
            pl.BlockSpec((Q_, D_), lambda kb: (0, 0)),
            pl.BlockSpec((KB_, D_), lambda kb: (kb, 0)),
        ],
        out_specs=[
            pl.BlockSpec((Q_, KB_), lambda kb: (0, kb)),
            pl.BlockSpec((1, Q_, CPB_), lambda kb: (kb, 0, 0)),
        ],
        out_shape=[
            jax.ShapeDtypeStruct((Q_, KPAD_), jnp.float32),
            jax.ShapeDtypeStruct((NB_, Q_, CPB_), jnp.float32),
        ],
        scratch_shapes=[pltpu.VMEM((Q_, D_), jnp.float32)],
    )(queries, keys_pad)


def kernel(queries, drug_h, keys, w_rel, Wg_w, Wg_b):
    keys_pad = jnp.concatenate(
        [keys, jnp.zeros((KPAD_ - K_, D_), jnp.float32)], axis=0)
    sim, m3 = _sim_stage(queries, keys_pad)
    m = jnp.transpose(m3, (1, 0, 2)).reshape(Q_, NCH_)

    _, chunk_ids = jax.lax.top_k(m, TK_)
    simr = sim.reshape(Q_, NCH_, CH_)
    cand = jnp.take_along_axis(simr, chunk_ids[:, :, None], axis=1)
    cand = cand.reshape(Q_, TK_ * CH_)
    topv, pos = jax.lax.top_k(cand, TK_)
    sel_chunk = jnp.take_along_axis(chunk_ids, pos // CH_, axis=1)
    key_idx = sel_chunk * CH_ + pos % CH_
    coef = jax.nn.softmax(topv, axis=1)
    embeds = jnp.take(keys, key_idx, axis=0)
    proto = jnp.sum(coef[:, :, None] * embeds, axis=1)
    gate_in = jnp.concatenate([queries, proto], axis=1)
    gate = jax.nn.sigmoid(gate_in @ Wg_w.T + Wg_b)
    h_dis = gate * queries + (1.0 - gate) * proto
    score = jax.nn.sigmoid(jnp.sum(drug_h * w_rel[None, :] * h_dis, axis=1))
    return score
